# Initial kernel scaffold; baseline (speedup 1.0000x reference)
import jax
import jax.numpy as jnp
from jax.experimental import pallas as pl


def kernel(x, edge_index, batch_index, patient_encoder_output, edgebindex, W1, b1, g1, beta1, W2, b2, g2, beta2, W_fc, b_fc, g_e, beta_e):
    raise NotImplementedError("write your pallas kernel here")


# speedup vs baseline: 2.5643x; 2.5643x over previous
import functools

import jax
import jax.numpy as jnp
from jax import lax
from jax.experimental import pallas as pl
from jax.experimental.pallas import tpu as pltpu
from jax.experimental.pallas import tpu_sc as plsc

N = 10000
N2 = N + 16
E = 160000
E_PAD = 163840
DH = 256
DHH = 128
P = 64
NC = 2
NS = 16
CHUNK = 128
ROWS_T = 624
ROWS_TAIL = N2 - ROWS_T * NS

_f32 = jnp.float32


def _sc_mesh():
    return plsc.VectorSubcoreMesh(core_axis_name="c", subcore_axis_name="s")


def _rows_copy(src_ref, dst_ref, s):
    r0 = pl.multiple_of(s * ROWS_T, 8)
    pltpu.sync_copy(src_ref.at[pl.ds(r0, ROWS_T)], dst_ref.at[pl.ds(r0, ROWS_T)])

    @pl.when(s == NS - 1)
    def _():
        t0 = ROWS_T * NS
        pltpu.sync_copy(src_ref.at[pl.ds(t0, ROWS_TAIL)],
                        dst_ref.at[pl.ds(t0, ROWS_TAIL)])


def _deg_body(dst_hbm, ones_hbm, zeros_hbm, deg_a, deg_b, idx_v, ones_v,
              shared, sem):
    c = lax.axis_index("c")
    s = lax.axis_index("s")
    _rows_copy(zeros_hbm, shared, s)
    pltpu.sync_copy(ones_hbm, ones_v)
    plsc.subcore_barrier()

    wid = c * NS + s
    n_chunks = E_PAD // (NC * NS * CHUNK)

    def chunk(j, _):
        base = pl.multiple_of(wid * (E_PAD // (NC * NS)) + j * CHUNK, 8)
        pltpu.sync_copy(dst_hbm.at[pl.ds(base, CHUNK)], idx_v)
        pltpu.sync_copy(ones_v, shared.at[idx_v], add=True)
        return 0

    lax.fori_loop(0, n_chunks, chunk, 0)
    plsc.subcore_barrier()

    @pl.when(c == 0)
    def _():
        _rows_copy(shared, deg_a, s)

    @pl.when(c == 1)
    def _():
        _rows_copy(shared, deg_b, s)


@jax.jit
def _deg_call(dst, ones, zeros128):
    k = functools.partial(
        pl.kernel,
        mesh=_sc_mesh(),
        out_type=[jax.ShapeDtypeStruct((N2, DHH), _f32),
                  jax.ShapeDtypeStruct((N2, DHH), _f32)],
        scratch_types=[pltpu.VMEM((CHUNK,), jnp.int32),
                       pltpu.VMEM((CHUNK, DHH), _f32),
                       pltpu.VMEM_SHARED((N2, DHH), _f32),
                       pltpu.SemaphoreType.DMA],
    )(_deg_body)
    return k(dst, ones, zeros128)


def _conv_body(src_hbm, dst_hbm, ta_hbm, tb_hbm, zeros_hbm, out_a, out_b,
               src_v, dst_v, rows_v, shared, sem):
    c = lax.axis_index("c")
    s = lax.axis_index("s")
    _rows_copy(zeros_hbm, shared, s)
    plsc.subcore_barrier()

    e_per_tile = E_PAD // NS
    n_chunks = e_per_tile // CHUNK

    def chunk(j, tbl):
        base = pl.multiple_of(s * e_per_tile + j * CHUNK, 8)
        pltpu.sync_copy(src_hbm.at[pl.ds(base, CHUNK)], src_v)
        pltpu.sync_copy(dst_hbm.at[pl.ds(base, CHUNK)], dst_v)
        pltpu.async_copy(tbl.at[src_v], rows_v, sem).wait()
        pltpu.sync_copy(rows_v, shared.at[dst_v], add=True)

    @pl.when(c == 0)
    def _():
        lax.fori_loop(0, n_chunks, lambda j, _: (chunk(j, ta_hbm), 0)[1], 0)

    @pl.when(c == 1)
    def _():
        lax.fori_loop(0, n_chunks, lambda j, _: (chunk(j, tb_hbm), 0)[1], 0)

    plsc.subcore_barrier()

    @pl.when(c == 0)
    def _():
        _rows_copy(shared, out_a, s)

    @pl.when(c == 1)
    def _():
        _rows_copy(shared, out_b, s)


@jax.jit
def _conv_call(src, dst, ta, tb, zeros128):
    k = functools.partial(
        pl.kernel,
        mesh=_sc_mesh(),
        out_type=[jax.ShapeDtypeStruct((N2, DHH), _f32),
                  jax.ShapeDtypeStruct((N2, DHH), _f32)],
        scratch_types=[pltpu.VMEM((CHUNK,), jnp.int32),
                       pltpu.VMEM((CHUNK,), jnp.int32),
                       pltpu.VMEM((CHUNK, DHH), _f32),
                       pltpu.VMEM_SHARED((N2, DHH), _f32),
                       pltpu.SemaphoreType.DMA],
    )(_conv_body)
    return k(src, dst, ta, tb, zeros128)


def _edge_head_body(src_hbm, dst_hbm, ebi_hbm, ps_h, pd_h, pp_h, iota_hbm,
                    g_hbm, i1, i2, i3, ident, a_v, b_v, c_v, shared, sem):
    c = lax.axis_index("c")
    s = lax.axis_index("s")
    wid = c * NS + s
    e_per_w = E_PAD // (NC * NS)
    n_chunks = e_per_w // CHUNK
    pltpu.sync_copy(iota_hbm.at[s], ident)
    r0 = s * CHUNK

    def chunk(j, _):
        base = pl.multiple_of(wid * e_per_w + j * CHUNK, 8)
        pltpu.sync_copy(src_hbm.at[pl.ds(base, CHUNK)], i1)
        pltpu.sync_copy(dst_hbm.at[pl.ds(base, CHUNK)], i2)
        pltpu.sync_copy(ebi_hbm.at[pl.ds(base, CHUNK)], i3)
        pltpu.async_copy(ps_h.at[i1], a_v, sem).wait()
        pltpu.async_copy(pd_h.at[i2], b_v, sem).wait()
        pltpu.async_copy(pp_h.at[i3], c_v, sem).wait()
        pltpu.sync_copy(a_v, shared.at[pl.ds(r0, CHUNK)])
        pltpu.sync_copy(b_v, shared.at[ident], add=True)
        pltpu.sync_copy(c_v, shared.at[ident], add=True)
        pltpu.sync_copy(shared.at[pl.ds(r0, CHUNK)], g_hbm.at[pl.ds(base, CHUNK)])
        return 0

    lax.fori_loop(0, n_chunks, chunk, 0)


@jax.jit
def _edge_head_call(src_p, dst_p, ebi_p, ps128, pd128, pp128, iota2d):
    k = functools.partial(
        pl.kernel,
        mesh=_sc_mesh(),
        out_type=jax.ShapeDtypeStruct((E_PAD, DHH), _f32),
        scratch_types=[pltpu.VMEM((CHUNK,), jnp.int32),
                       pltpu.VMEM((CHUNK,), jnp.int32),
                       pltpu.VMEM((CHUNK,), jnp.int32),
                       pltpu.VMEM((CHUNK,), jnp.int32),
                       pltpu.VMEM((CHUNK, DHH), _f32),
                       pltpu.VMEM((CHUNK, DHH), _f32),
                       pltpu.VMEM((CHUNK, DHH), _f32),
                       pltpu.VMEM_SHARED((NS * CHUNK, DHH), _f32),
                       pltpu.SemaphoreType.DMA],
    )(_edge_head_body)
    return k(src_p, dst_p, ebi_p, ps128, pd128, pp128, iota2d)


def _bn_cols(h, g, beta):
    mu = jnp.mean(h, axis=0, keepdims=True)
    var = jnp.mean((h - mu) ** 2, axis=0, keepdims=True)
    return g[None, :] * (h - mu) * lax.rsqrt(var + 1e-5) + beta[None, :]


def _stage1_body(deg_a, deg_b, x, w1, t1a, t1b, dis_out):
    deg = deg_a[...][:N, :1] + deg_b[...][:N, :1]
    dis = jnp.where(deg > 0, lax.rsqrt(deg), 0.0)
    m = jnp.dot(x[...], w1[...], preferred_element_type=_f32)
    t = dis * m
    t1a[...] = t[:, :DHH]
    t1b[...] = t[:, DHH:]
    dis_out[...] = dis


@jax.jit
def _stage1_call(deg_a, deg_b, x, w1):
    return pl.pallas_call(
        _stage1_body,
        out_shape=[jax.ShapeDtypeStruct((N, DHH), _f32),
                   jax.ShapeDtypeStruct((N, DHH), _f32),
                   jax.ShapeDtypeStruct((N, 1), _f32)],
    )(deg_a, deg_b, x, w1)


def _stage2_body(sa, sb, dis, b, g, beta, w2, out_a, out_b):
    h = jnp.concatenate([sa[...][:N], sb[...][:N]], axis=1)
    h = jnp.maximum(dis[...] * h + b[...][None, :], 0.0)
    hn = _bn_cols(h, g[...], beta[...])
    t = dis[...] * jnp.dot(hn, w2[...], preferred_element_type=_f32)
    out_a[...] = t[:, :DHH]
    out_b[...] = t[:, DHH:]


@jax.jit
def _stage2_call(sa, sb, dis, b1, g1, beta1, w2):
    return pl.pallas_call(
        _stage2_body,
        out_shape=[jax.ShapeDtypeStruct((N, DHH), _f32),
                   jax.ShapeDtypeStruct((N, DHH), _f32)],
    )(sa, sb, dis, b1, g1, beta1, w2)


def _stage3_body(sa, sb, dis, b, g, beta, wsrc, wdst, pat, wp,
                 ps_o, pd_o, pp_o):
    h = jnp.concatenate([sa[...][:N], sb[...][:N]], axis=1)
    h = jnp.maximum(dis[...] * h + b[...][None, :], 0.0)
    hn = _bn_cols(h, g[...], beta[...])
    ps_o[...] = jnp.dot(hn, wsrc[...], preferred_element_type=_f32)
    pd_o[...] = jnp.dot(hn, wdst[...], preferred_element_type=_f32)
    pp_o[...] = jnp.dot(pat[...], wp[...], preferred_element_type=_f32)


@jax.jit
def _stage3_call(sa, sb, dis, b2, g2, beta2, wsrc128, wdst128, pat, wp128):
    return pl.pallas_call(
        _stage3_body,
        out_shape=[jax.ShapeDtypeStruct((N, DHH), _f32),
                   jax.ShapeDtypeStruct((N, DHH), _f32),
                   jax.ShapeDtypeStruct((P, DHH), _f32)],
    )(sa, sb, dis, b2, g2, beta2, wsrc128, wdst128, pat, wp128)


K4 = 16
EC = E_PAD // K4


def _stage4a_body(g, bfc, e_o, s1_o, s2_o):
    k = pl.program_id(0)
    row = k * EC + lax.broadcasted_iota(jnp.int32, (EC, 1), 0)
    valid = (row < E).astype(_f32)
    e = g[...][:, :2] + bfc[...][None, :]
    e_o[...] = e
    ev = e * valid
    s1_o[...] = jnp.sum(ev, axis=0)[None, None, :]
    s2_o[...] = jnp.sum(ev * e, axis=0)[None, None, :]


@jax.jit
def _stage4a_call(g, bfc):
    return pl.pallas_call(
        _stage4a_body,
        grid=(K4,),
        in_specs=[pl.BlockSpec((EC, DHH), lambda k: (k, 0)),
                  pl.BlockSpec((2,), lambda k: (0,))],
        out_specs=[pl.BlockSpec((EC, 2), lambda k: (k, 0)),
                   pl.BlockSpec((1, 1, 2), lambda k: (k, 0, 0)),
                   pl.BlockSpec((1, 1, 2), lambda k: (k, 0, 0))],
        out_shape=[jax.ShapeDtypeStruct((E_PAD, 2), _f32),
                   jax.ShapeDtypeStruct((K4, 1, 2), _f32),
                   jax.ShapeDtypeStruct((K4, 1, 2), _f32)],
    )(g, bfc)


def _stage4b_body(e, s1, s2, ge, betae, out):
    mu = jnp.sum(s1[...][:, 0, :], axis=0)[None, :] / E
    var = jnp.sum(s2[...][:, 0, :], axis=0)[None, :] / E - mu * mu
    out[...] = (ge[...][None, :] * (e[...] - mu) * lax.rsqrt(var + 1e-5)
                + betae[...][None, :])


@jax.jit
def _stage4b_call(e, s1, s2, ge, betae):
    return pl.pallas_call(
        _stage4b_body,
        grid=(K4,),
        in_specs=[pl.BlockSpec((EC, 2), lambda k: (k, 0)),
                  pl.BlockSpec((K4, 1, 2), lambda k: (0, 0, 0)),
                  pl.BlockSpec((K4, 1, 2), lambda k: (0, 0, 0)),
                  pl.BlockSpec((2,), lambda k: (0,)),
                  pl.BlockSpec((2,), lambda k: (0,))],
        out_specs=pl.BlockSpec((EC, 2), lambda k: (k, 0)),
        out_shape=jax.ShapeDtypeStruct((E_PAD, 2), _f32),
    )(e, s1, s2, ge, betae)


def kernel(x, edge_index, batch_index, patient_encoder_output, edgebindex,
           W1, b1, g1, beta1, W2, b2, g2, beta2, W_fc, b_fc, g_e, beta_e):
    pad0 = jnp.zeros((E_PAD - E,), jnp.int32)
    padn = jnp.full((E_PAD - E,), N, jnp.int32)
    src = jnp.concatenate([edge_index[0], pad0])
    dst = jnp.concatenate([edge_index[1], padn])
    ebi = jnp.concatenate([edgebindex, pad0])
    ones = jnp.ones((CHUNK, DHH), _f32)
    zeros128 = jnp.zeros((N2, DHH), _f32)
    padc = jnp.zeros((DH, DHH - 2), _f32)
    wsrc128 = jnp.concatenate([W_fc[:DH], padc], axis=1)
    wdst128 = jnp.concatenate([W_fc[DH:2 * DH], padc], axis=1)
    wp128 = jnp.concatenate([W_fc[2 * DH:], padc[:P + P]], axis=1)
    iota2d = jnp.arange(NS * CHUNK, dtype=jnp.int32).reshape(NS, CHUNK)

    deg_a, deg_b = _deg_call(dst, ones, zeros128)
    t1a, t1b, dis = _stage1_call(deg_a, deg_b, x, W1)
    s1a, s1b = _conv_call(src, dst, t1a, t1b, zeros128)
    t2a, t2b = _stage2_call(s1a, s1b, dis, b1, g1, beta1, W2)
    s2a, s2b = _conv_call(src, dst, t2a, t2b, zeros128)
    ps128, pd128, pp128 = _stage3_call(s2a, s2b, dis, b2, g2, beta2,
                                       wsrc128, wdst128,
                                       patient_encoder_output, wp128)
    ge_rows = _edge_head_call(src, dst, ebi, ps128, pd128, pp128, iota2d)
    e, s1, s2 = _stage4a_call(ge_rows, b_fc)
    return _stage4b_call(e, s1, s2, g_e, beta_e)[:E]


# speedup vs baseline: 3.1239x; 1.2182x over previous
import functools

import jax
import jax.numpy as jnp
from jax import lax
from jax.experimental import pallas as pl
from jax.experimental.pallas import tpu as pltpu
from jax.experimental.pallas import tpu_sc as plsc

N = 10000
N2 = N + 16
E = 160000
E_PAD = 163840
DH = 256
DHH = 128
P = 64
NC = 2
NS = 16
CHUNK = 128
ROWS_T = 624
ROWS_TAIL = N2 - ROWS_T * NS

_f32 = jnp.float32


def _sc_mesh():
    return plsc.VectorSubcoreMesh(core_axis_name="c", subcore_axis_name="s")


def _rows_copy(src_ref, dst_ref, s):
    r0 = pl.multiple_of(s * ROWS_T, 8)
    pltpu.sync_copy(src_ref.at[pl.ds(r0, ROWS_T)], dst_ref.at[pl.ds(r0, ROWS_T)])

    @pl.when(s == NS - 1)
    def _():
        t0 = ROWS_T * NS
        pltpu.sync_copy(src_ref.at[pl.ds(t0, ROWS_TAIL)],
                        dst_ref.at[pl.ds(t0, ROWS_TAIL)])


def _deg_body(dst_hbm, ones_hbm, zeros_hbm, deg_a, deg_b, idx_v, ones_v,
              shared, sem):
    c = lax.axis_index("c")
    s = lax.axis_index("s")
    _rows_copy(zeros_hbm, shared, s)
    pltpu.sync_copy(ones_hbm, ones_v)
    plsc.subcore_barrier()

    wid = c * NS + s
    n_chunks = E_PAD // (NC * NS * CHUNK)

    def chunk(j, _):
        base = pl.multiple_of(wid * (E_PAD // (NC * NS)) + j * CHUNK, 8)
        pltpu.sync_copy(dst_hbm.at[pl.ds(base, CHUNK)], idx_v)
        pltpu.sync_copy(ones_v, shared.at[idx_v], add=True)
        return 0

    lax.fori_loop(0, n_chunks, chunk, 0)
    plsc.subcore_barrier()

    @pl.when(c == 0)
    def _():
        _rows_copy(shared, deg_a, s)

    @pl.when(c == 1)
    def _():
        _rows_copy(shared, deg_b, s)


@jax.jit
def _deg_call(dst, ones, zeros128):
    k = functools.partial(
        pl.kernel,
        mesh=_sc_mesh(),
        out_type=[jax.ShapeDtypeStruct((N2, DHH), _f32),
                  jax.ShapeDtypeStruct((N2, DHH), _f32)],
        scratch_types=[pltpu.VMEM((CHUNK,), jnp.int32),
                       pltpu.VMEM((CHUNK, DHH), _f32),
                       pltpu.VMEM_SHARED((N2, DHH), _f32),
                       pltpu.SemaphoreType.DMA],
    )(_deg_body)
    return k(dst, ones, zeros128)


def _conv_body(src_hbm, dst_hbm, ta_hbm, tb_hbm, zeros_hbm, out_a, out_b,
               src_v, dst_v, rows_v, shared, sem):
    c = lax.axis_index("c")
    s = lax.axis_index("s")
    _rows_copy(zeros_hbm, shared, s)
    plsc.subcore_barrier()

    e_per_tile = E_PAD // NS
    n_chunks = e_per_tile // CHUNK

    def chunk(j, tbl):
        base = pl.multiple_of(s * e_per_tile + j * CHUNK, 8)
        pltpu.sync_copy(src_hbm.at[pl.ds(base, CHUNK)], src_v)
        pltpu.sync_copy(dst_hbm.at[pl.ds(base, CHUNK)], dst_v)
        pltpu.async_copy(tbl.at[src_v], rows_v, sem).wait()
        pltpu.sync_copy(rows_v, shared.at[dst_v], add=True)

    @pl.when(c == 0)
    def _():
        lax.fori_loop(0, n_chunks, lambda j, _: (chunk(j, ta_hbm), 0)[1], 0)

    @pl.when(c == 1)
    def _():
        lax.fori_loop(0, n_chunks, lambda j, _: (chunk(j, tb_hbm), 0)[1], 0)

    plsc.subcore_barrier()

    @pl.when(c == 0)
    def _():
        _rows_copy(shared, out_a, s)

    @pl.when(c == 1)
    def _():
        _rows_copy(shared, out_b, s)


@jax.jit
def _conv_call(src, dst, ta, tb, zeros128):
    k = functools.partial(
        pl.kernel,
        mesh=_sc_mesh(),
        out_type=[jax.ShapeDtypeStruct((N2, DHH), _f32),
                  jax.ShapeDtypeStruct((N2, DHH), _f32)],
        scratch_types=[pltpu.VMEM((CHUNK,), jnp.int32),
                       pltpu.VMEM((CHUNK,), jnp.int32),
                       pltpu.VMEM((CHUNK, DHH), _f32),
                       pltpu.VMEM_SHARED((N2, DHH), _f32),
                       pltpu.SemaphoreType.DMA],
    )(_conv_body)
    return k(src, dst, ta, tb, zeros128)


def _edge_head_body(src_hbm, dst_hbm, ps_h, pd_h, iota_hbm,
                    g_hbm, i1, i2, ident, a_v, b_v, shared, sem1, sem2):
    c = lax.axis_index("c")
    s = lax.axis_index("s")
    wid = c * NS + s
    e_per_w = E_PAD // (NC * NS)
    n_chunks = e_per_w // CHUNK
    pltpu.sync_copy(iota_hbm.at[s], ident)
    r0 = s * CHUNK

    def chunk(j, _):
        base = pl.multiple_of(wid * e_per_w + j * CHUNK, 8)
        pltpu.sync_copy(src_hbm.at[pl.ds(base, CHUNK)], i1)
        pltpu.sync_copy(dst_hbm.at[pl.ds(base, CHUNK)], i2)
        cp1 = pltpu.async_copy(ps_h.at[i1], a_v, sem1)
        cp2 = pltpu.async_copy(pd_h.at[i2], b_v, sem2)
        cp1.wait()
        cp2.wait()
        pltpu.sync_copy(a_v, shared.at[pl.ds(r0, CHUNK)])
        pltpu.sync_copy(b_v, shared.at[ident], add=True)
        pltpu.sync_copy(shared.at[pl.ds(r0, CHUNK)], g_hbm.at[pl.ds(base, CHUNK)])
        return 0

    lax.fori_loop(0, n_chunks, chunk, 0)


@jax.jit
def _edge_head_call(src_p, dst_p, ps128, pd128, iota2d):
    k = functools.partial(
        pl.kernel,
        mesh=_sc_mesh(),
        out_type=jax.ShapeDtypeStruct((E_PAD, DHH), _f32),
        scratch_types=[pltpu.VMEM((CHUNK,), jnp.int32),
                       pltpu.VMEM((CHUNK,), jnp.int32),
                       pltpu.VMEM((CHUNK,), jnp.int32),
                       pltpu.VMEM((CHUNK, DHH), _f32),
                       pltpu.VMEM((CHUNK, DHH), _f32),
                       pltpu.VMEM_SHARED((NS * CHUNK, DHH), _f32),
                       pltpu.SemaphoreType.DMA,
                       pltpu.SemaphoreType.DMA],
    )(_edge_head_body)
    return k(src_p, dst_p, ps128, pd128, iota2d)


def _bn_cols(h, g, beta):
    mu = jnp.mean(h, axis=0, keepdims=True)
    var = jnp.mean((h - mu) ** 2, axis=0, keepdims=True)
    return g[None, :] * (h - mu) * lax.rsqrt(var + 1e-5) + beta[None, :]


def _stage1_body(deg_a, deg_b, x, w1, t1a, t1b, dis_out):
    deg = deg_a[...][:N, :1] + deg_b[...][:N, :1]
    dis = jnp.where(deg > 0, lax.rsqrt(deg), 0.0)
    m = jnp.dot(x[...], w1[...], preferred_element_type=_f32)
    t = dis * m
    t1a[...] = t[:, :DHH]
    t1b[...] = t[:, DHH:]
    dis_out[...] = dis


@jax.jit
def _stage1_call(deg_a, deg_b, x, w1):
    return pl.pallas_call(
        _stage1_body,
        out_shape=[jax.ShapeDtypeStruct((N, DHH), _f32),
                   jax.ShapeDtypeStruct((N, DHH), _f32),
                   jax.ShapeDtypeStruct((N, 1), _f32)],
    )(deg_a, deg_b, x, w1)


def _stage2_body(sa, sb, dis, b, g, beta, w2, out_a, out_b):
    h = jnp.concatenate([sa[...][:N], sb[...][:N]], axis=1)
    h = jnp.maximum(dis[...] * h + b[...][None, :], 0.0)
    hn = _bn_cols(h, g[...], beta[...])
    t = dis[...] * jnp.dot(hn, w2[...], preferred_element_type=_f32)
    out_a[...] = t[:, :DHH]
    out_b[...] = t[:, DHH:]


@jax.jit
def _stage2_call(sa, sb, dis, b1, g1, beta1, w2):
    return pl.pallas_call(
        _stage2_body,
        out_shape=[jax.ShapeDtypeStruct((N, DHH), _f32),
                   jax.ShapeDtypeStruct((N, DHH), _f32)],
    )(sa, sb, dis, b1, g1, beta1, w2)


def _stage3_body(sa, sb, dis, b, g, beta, wsrc, wdst, pat, wp,
                 ps_o, pd_o, pp_o):
    h = jnp.concatenate([sa[...][:N], sb[...][:N]], axis=1)
    h = jnp.maximum(dis[...] * h + b[...][None, :], 0.0)
    hn = _bn_cols(h, g[...], beta[...])
    ps_o[...] = jnp.dot(hn, wsrc[...], preferred_element_type=_f32)
    pd_o[...] = jnp.dot(hn, wdst[...], preferred_element_type=_f32)
    pp_o[...] = jnp.dot(pat[...], wp[...], preferred_element_type=_f32)


@jax.jit
def _stage3_call(sa, sb, dis, b2, g2, beta2, wsrc128, wdst128, pat, wp128):
    return pl.pallas_call(
        _stage3_body,
        out_shape=[jax.ShapeDtypeStruct((N, DHH), _f32),
                   jax.ShapeDtypeStruct((N, DHH), _f32),
                   jax.ShapeDtypeStruct((P, DHH), _f32)],
    )(sa, sb, dis, b2, g2, beta2, wsrc128, wdst128, pat, wp128)


K4 = 16
EC = E_PAD // K4


def _stage4a_body(g, bfc, ebi, pp, e_o, s1_o, s2_o):
    k = pl.program_id(0)
    row = k * EC + lax.broadcasted_iota(jnp.int32, (EC, 1), 0)
    valid = (row < E).astype(_f32)
    onehot = (ebi[...] == lax.broadcasted_iota(jnp.int32, (1, P), 1)
              ).astype(_f32)
    ppe = jnp.dot(onehot, pp[...][:, :2], preferred_element_type=_f32)
    e = g[...][:, :2] + ppe + bfc[...][None, :]
    e_o[...] = e
    ev = e * valid
    s1_o[...] = jnp.sum(ev, axis=0)[None, None, :]
    s2_o[...] = jnp.sum(ev * e, axis=0)[None, None, :]


@jax.jit
def _stage4a_call(g, bfc, ebi2d, pp128):
    return pl.pallas_call(
        _stage4a_body,
        grid=(K4,),
        in_specs=[pl.BlockSpec((EC, DHH), lambda k: (k, 0)),
                  pl.BlockSpec((2,), lambda k: (0,)),
                  pl.BlockSpec((EC, 1), lambda k: (k, 0)),
                  pl.BlockSpec((P, DHH), lambda k: (0, 0))],
        out_specs=[pl.BlockSpec((EC, 2), lambda k: (k, 0)),
                   pl.BlockSpec((1, 1, 2), lambda k: (k, 0, 0)),
                   pl.BlockSpec((1, 1, 2), lambda k: (k, 0, 0))],
        out_shape=[jax.ShapeDtypeStruct((E_PAD, 2), _f32),
                   jax.ShapeDtypeStruct((K4, 1, 2), _f32),
                   jax.ShapeDtypeStruct((K4, 1, 2), _f32)],
    )(g, bfc, ebi2d, pp128)


def _stage4b_body(e, s1, s2, ge, betae, out):
    mu = jnp.sum(s1[...][:, 0, :], axis=0)[None, :] / E
    var = jnp.sum(s2[...][:, 0, :], axis=0)[None, :] / E - mu * mu
    out[...] = (ge[...][None, :] * (e[...] - mu) * lax.rsqrt(var + 1e-5)
                + betae[...][None, :])


@jax.jit
def _stage4b_call(e, s1, s2, ge, betae):
    return pl.pallas_call(
        _stage4b_body,
        grid=(K4,),
        in_specs=[pl.BlockSpec((EC, 2), lambda k: (k, 0)),
                  pl.BlockSpec((K4, 1, 2), lambda k: (0, 0, 0)),
                  pl.BlockSpec((K4, 1, 2), lambda k: (0, 0, 0)),
                  pl.BlockSpec((2,), lambda k: (0,)),
                  pl.BlockSpec((2,), lambda k: (0,))],
        out_specs=pl.BlockSpec((EC, 2), lambda k: (k, 0)),
        out_shape=jax.ShapeDtypeStruct((E_PAD, 2), _f32),
    )(e, s1, s2, ge, betae)


def kernel(x, edge_index, batch_index, patient_encoder_output, edgebindex,
           W1, b1, g1, beta1, W2, b2, g2, beta2, W_fc, b_fc, g_e, beta_e):
    pad0 = jnp.zeros((E_PAD - E,), jnp.int32)
    padn = jnp.full((E_PAD - E,), N, jnp.int32)
    src = jnp.concatenate([edge_index[0], pad0])
    dst = jnp.concatenate([edge_index[1], padn])
    dst_eh = jnp.concatenate([edge_index[1], pad0])
    ebi2d = jnp.concatenate([edgebindex, pad0])[:, None]
    ones = jnp.ones((CHUNK, DHH), _f32)
    zeros128 = jnp.zeros((N2, DHH), _f32)
    padc = jnp.zeros((DH, DHH - 2), _f32)
    wsrc128 = jnp.concatenate([W_fc[:DH], padc], axis=1)
    wdst128 = jnp.concatenate([W_fc[DH:2 * DH], padc], axis=1)
    wp128 = jnp.concatenate([W_fc[2 * DH:], padc[:P + P]], axis=1)
    iota2d = jnp.arange(NS * CHUNK, dtype=jnp.int32).reshape(NS, CHUNK)

    deg_a, deg_b = _deg_call(dst, ones, zeros128)
    t1a, t1b, dis = _stage1_call(deg_a, deg_b, x, W1)
    s1a, s1b = _conv_call(src, dst, t1a, t1b, zeros128)
    t2a, t2b = _stage2_call(s1a, s1b, dis, b1, g1, beta1, W2)
    s2a, s2b = _conv_call(src, dst, t2a, t2b, zeros128)
    ps128, pd128, pp128 = _stage3_call(s2a, s2b, dis, b2, g2, beta2,
                                       wsrc128, wdst128,
                                       patient_encoder_output, wp128)
    ge_rows = _edge_head_call(src, dst_eh, ps128, pd128, iota2d)
    e, s1, s2 = _stage4a_call(ge_rows, b_fc, ebi2d, pp128)
    return _stage4b_call(e, s1, s2, g_e, beta_e)[:E]


# speedup vs baseline: 3.3990x; 1.0881x over previous
import functools

import jax
import jax.numpy as jnp
from jax import lax
from jax.experimental import pallas as pl
from jax.experimental.pallas import tpu as pltpu
from jax.experimental.pallas import tpu_sc as plsc

N = 10000
N2 = N + 16
E = 160000
E_PAD = 163840
DH = 256
DHH = 128
P = 64
NC = 2
NS = 16
CHUNK = 128
ROWS_T = 624
ROWS_TAIL = N2 - ROWS_T * NS

_f32 = jnp.float32


def _sc_mesh():
    return plsc.VectorSubcoreMesh(core_axis_name="c", subcore_axis_name="s")


def _rows_copy(src_ref, dst_ref, s):
    r0 = pl.multiple_of(s * ROWS_T, 8)
    pltpu.sync_copy(src_ref.at[pl.ds(r0, ROWS_T)], dst_ref.at[pl.ds(r0, ROWS_T)])

    @pl.when(s == NS - 1)
    def _():
        t0 = ROWS_T * NS
        pltpu.sync_copy(src_ref.at[pl.ds(t0, ROWS_TAIL)],
                        dst_ref.at[pl.ds(t0, ROWS_TAIL)])


def _deg_body(dst_hbm, ones_hbm, zeros_hbm, deg_a, deg_b, idx_v, ones_v,
              shared, sem):
    c = lax.axis_index("c")
    s = lax.axis_index("s")
    _rows_copy(zeros_hbm, shared, s)
    pltpu.sync_copy(ones_hbm, ones_v)
    plsc.subcore_barrier()

    wid = c * NS + s
    n_chunks = E_PAD // (NC * NS * CHUNK)

    def chunk(j, _):
        base = pl.multiple_of(wid * (E_PAD // (NC * NS)) + j * CHUNK, 8)
        pltpu.sync_copy(dst_hbm.at[pl.ds(base, CHUNK)], idx_v)
        pltpu.sync_copy(ones_v, shared.at[idx_v], add=True)
        return 0

    lax.fori_loop(0, n_chunks, chunk, 0)
    plsc.subcore_barrier()

    @pl.when(c == 0)
    def _():
        _rows_copy(shared, deg_a, s)

    @pl.when(c == 1)
    def _():
        _rows_copy(shared, deg_b, s)


@jax.jit
def _deg_call(dst, ones, zeros128):
    k = functools.partial(
        pl.kernel,
        mesh=_sc_mesh(),
        out_type=[jax.ShapeDtypeStruct((N2, DHH), _f32),
                  jax.ShapeDtypeStruct((N2, DHH), _f32)],
        scratch_types=[pltpu.VMEM((CHUNK,), jnp.int32),
                       pltpu.VMEM((CHUNK, DHH), _f32),
                       pltpu.VMEM_SHARED((N2, DHH), _f32),
                       pltpu.SemaphoreType.DMA],
    )(_deg_body)
    return k(dst, ones, zeros128)


def _conv_body(src_hbm, dst_hbm, ta_hbm, tb_hbm, zeros_hbm, out_a, out_b,
               src_a, dst_a, src_b, dst_b, rows_a, rows_b, shared,
               sem_a, sem_b):
    c = lax.axis_index("c")
    s = lax.axis_index("s")
    _rows_copy(zeros_hbm, shared, s)
    plsc.subcore_barrier()

    e_per_tile = E_PAD // NS
    n_chunks = e_per_tile // CHUNK
    n_pairs = n_chunks // 2

    def sweep(tbl):
        def load_issue(j, src_v, dst_v, rows_v, sem):
            base = pl.multiple_of(s * e_per_tile + j * CHUNK, 8)
            pltpu.sync_copy(src_hbm.at[pl.ds(base, CHUNK)], src_v)
            pltpu.sync_copy(dst_hbm.at[pl.ds(base, CHUNK)], dst_v)
            return pltpu.async_copy(tbl.at[src_v], rows_v, sem)

        def pair(k, _):
            cp_a = load_issue(2 * k, src_a, dst_a, rows_a, sem_a)
            cp_b = load_issue(2 * k + 1, src_b, dst_b, rows_b, sem_b)
            cp_a.wait()
            pltpu.sync_copy(rows_a, shared.at[dst_a], add=True)
            cp_b.wait()
            pltpu.sync_copy(rows_b, shared.at[dst_b], add=True)
            return 0

        lax.fori_loop(0, n_pairs, pair, 0)

    @pl.when(c == 0)
    def _():
        sweep(ta_hbm)

    @pl.when(c == 1)
    def _():
        sweep(tb_hbm)

    plsc.subcore_barrier()

    @pl.when(c == 0)
    def _():
        _rows_copy(shared, out_a, s)

    @pl.when(c == 1)
    def _():
        _rows_copy(shared, out_b, s)


@jax.jit
def _conv_call(src, dst, ta, tb, zeros128):
    k = functools.partial(
        pl.kernel,
        mesh=_sc_mesh(),
        out_type=[jax.ShapeDtypeStruct((N2, DHH), _f32),
                  jax.ShapeDtypeStruct((N2, DHH), _f32)],
        scratch_types=[pltpu.VMEM((CHUNK,), jnp.int32),
                       pltpu.VMEM((CHUNK,), jnp.int32),
                       pltpu.VMEM((CHUNK,), jnp.int32),
                       pltpu.VMEM((CHUNK,), jnp.int32),
                       pltpu.VMEM((CHUNK, DHH), _f32),
                       pltpu.VMEM((CHUNK, DHH), _f32),
                       pltpu.VMEM_SHARED((N2, DHH), _f32),
                       pltpu.SemaphoreType.DMA,
                       pltpu.SemaphoreType.DMA],
    )(_conv_body)
    return k(src, dst, ta, tb, zeros128)


def _edge_head_body(src_hbm, dst_hbm, ps_h, pd_h, iota_hbm,
                    g_hbm, i1, i2, ident, a_v, b_v, shared, sem1, sem2):
    c = lax.axis_index("c")
    s = lax.axis_index("s")
    wid = c * NS + s
    e_per_w = E_PAD // (NC * NS)
    n_chunks = e_per_w // CHUNK
    pltpu.sync_copy(iota_hbm.at[s], ident)
    r0 = s * CHUNK

    def chunk(j, _):
        base = pl.multiple_of(wid * e_per_w + j * CHUNK, 8)
        pltpu.sync_copy(src_hbm.at[pl.ds(base, CHUNK)], i1)
        pltpu.sync_copy(dst_hbm.at[pl.ds(base, CHUNK)], i2)
        cp1 = pltpu.async_copy(ps_h.at[i1], a_v, sem1)
        cp2 = pltpu.async_copy(pd_h.at[i2], b_v, sem2)
        cp1.wait()
        cp2.wait()
        pltpu.sync_copy(a_v, shared.at[pl.ds(r0, CHUNK)])
        pltpu.sync_copy(b_v, shared.at[ident], add=True)
        pltpu.sync_copy(shared.at[pl.ds(r0, CHUNK)], g_hbm.at[pl.ds(base, CHUNK)])
        return 0

    lax.fori_loop(0, n_chunks, chunk, 0)


@jax.jit
def _edge_head_call(src_p, dst_p, ps128, pd128, iota2d):
    k = functools.partial(
        pl.kernel,
        mesh=_sc_mesh(),
        out_type=jax.ShapeDtypeStruct((E_PAD, DHH), _f32),
        scratch_types=[pltpu.VMEM((CHUNK,), jnp.int32),
                       pltpu.VMEM((CHUNK,), jnp.int32),
                       pltpu.VMEM((CHUNK,), jnp.int32),
                       pltpu.VMEM((CHUNK, DHH), _f32),
                       pltpu.VMEM((CHUNK, DHH), _f32),
                       pltpu.VMEM_SHARED((NS * CHUNK, DHH), _f32),
                       pltpu.SemaphoreType.DMA,
                       pltpu.SemaphoreType.DMA],
    )(_edge_head_body)
    return k(src_p, dst_p, ps128, pd128, iota2d)


def _bn_cols(h, g, beta):
    mu = jnp.mean(h, axis=0, keepdims=True)
    var = jnp.mean((h - mu) ** 2, axis=0, keepdims=True)
    return g[None, :] * (h - mu) * lax.rsqrt(var + 1e-5) + beta[None, :]


def _stage1_body(deg_a, deg_b, x, w1, t1a, t1b, dis_out):
    deg = deg_a[...][:N, :1] + deg_b[...][:N, :1]
    dis = jnp.where(deg > 0, lax.rsqrt(deg), 0.0)
    m = jnp.dot(x[...], w1[...], preferred_element_type=_f32)
    t = dis * m
    t1a[...] = t[:, :DHH]
    t1b[...] = t[:, DHH:]
    dis_out[...] = dis


@jax.jit
def _stage1_call(deg_a, deg_b, x, w1):
    return pl.pallas_call(
        _stage1_body,
        out_shape=[jax.ShapeDtypeStruct((N, DHH), _f32),
                   jax.ShapeDtypeStruct((N, DHH), _f32),
                   jax.ShapeDtypeStruct((N, 1), _f32)],
    )(deg_a, deg_b, x, w1)


def _stage2_body(sa, sb, dis, b, g, beta, w2, out_a, out_b):
    h = jnp.concatenate([sa[...][:N], sb[...][:N]], axis=1)
    h = jnp.maximum(dis[...] * h + b[...][None, :], 0.0)
    hn = _bn_cols(h, g[...], beta[...])
    t = dis[...] * jnp.dot(hn, w2[...], preferred_element_type=_f32)
    out_a[...] = t[:, :DHH]
    out_b[...] = t[:, DHH:]


@jax.jit
def _stage2_call(sa, sb, dis, b1, g1, beta1, w2):
    return pl.pallas_call(
        _stage2_body,
        out_shape=[jax.ShapeDtypeStruct((N, DHH), _f32),
                   jax.ShapeDtypeStruct((N, DHH), _f32)],
    )(sa, sb, dis, b1, g1, beta1, w2)


def _stage3_body(sa, sb, dis, b, g, beta, wsrc, wdst, pat, wp,
                 ps_o, pd_o, pp_o):
    h = jnp.concatenate([sa[...][:N], sb[...][:N]], axis=1)
    h = jnp.maximum(dis[...] * h + b[...][None, :], 0.0)
    hn = _bn_cols(h, g[...], beta[...])
    ps_o[...] = jnp.dot(hn, wsrc[...], preferred_element_type=_f32)
    pd_o[...] = jnp.dot(hn, wdst[...], preferred_element_type=_f32)
    pp_o[...] = jnp.dot(pat[...], wp[...], preferred_element_type=_f32)


@jax.jit
def _stage3_call(sa, sb, dis, b2, g2, beta2, wsrc128, wdst128, pat, wp128):
    return pl.pallas_call(
        _stage3_body,
        out_shape=[jax.ShapeDtypeStruct((N, DHH), _f32),
                   jax.ShapeDtypeStruct((N, DHH), _f32),
                   jax.ShapeDtypeStruct((P, DHH), _f32)],
    )(sa, sb, dis, b2, g2, beta2, wsrc128, wdst128, pat, wp128)


K4 = 16
EC = E_PAD // K4


def _stage4a_body(g, bfc, ebi, pp, e_o, s1_o, s2_o):
    k = pl.program_id(0)
    row = k * EC + lax.broadcasted_iota(jnp.int32, (EC, 1), 0)
    valid = (row < E).astype(_f32)
    onehot = (ebi[...] == lax.broadcasted_iota(jnp.int32, (1, P), 1)
              ).astype(_f32)
    ppe = jnp.dot(onehot, pp[...][:, :2], preferred_element_type=_f32)
    e = g[...][:, :2] + ppe + bfc[...][None, :]
    e_o[...] = e
    ev = e * valid
    s1_o[...] = jnp.sum(ev, axis=0)[None, None, :]
    s2_o[...] = jnp.sum(ev * e, axis=0)[None, None, :]


@jax.jit
def _stage4a_call(g, bfc, ebi2d, pp128):
    return pl.pallas_call(
        _stage4a_body,
        grid=(K4,),
        in_specs=[pl.BlockSpec((EC, DHH), lambda k: (k, 0)),
                  pl.BlockSpec((2,), lambda k: (0,)),
                  pl.BlockSpec((EC, 1), lambda k: (k, 0)),
                  pl.BlockSpec((P, DHH), lambda k: (0, 0))],
        out_specs=[pl.BlockSpec((EC, 2), lambda k: (k, 0)),
                   pl.BlockSpec((1, 1, 2), lambda k: (k, 0, 0)),
                   pl.BlockSpec((1, 1, 2), lambda k: (k, 0, 0))],
        out_shape=[jax.ShapeDtypeStruct((E_PAD, 2), _f32),
                   jax.ShapeDtypeStruct((K4, 1, 2), _f32),
                   jax.ShapeDtypeStruct((K4, 1, 2), _f32)],
    )(g, bfc, ebi2d, pp128)


def _stage4b_body(e, s1, s2, ge, betae, out):
    mu = jnp.sum(s1[...][:, 0, :], axis=0)[None, :] / E
    var = jnp.sum(s2[...][:, 0, :], axis=0)[None, :] / E - mu * mu
    out[...] = (ge[...][None, :] * (e[...] - mu) * lax.rsqrt(var + 1e-5)
                + betae[...][None, :])


@jax.jit
def _stage4b_call(e, s1, s2, ge, betae):
    return pl.pallas_call(
        _stage4b_body,
        grid=(K4,),
        in_specs=[pl.BlockSpec((EC, 2), lambda k: (k, 0)),
                  pl.BlockSpec((K4, 1, 2), lambda k: (0, 0, 0)),
                  pl.BlockSpec((K4, 1, 2), lambda k: (0, 0, 0)),
                  pl.BlockSpec((2,), lambda k: (0,)),
                  pl.BlockSpec((2,), lambda k: (0,))],
        out_specs=pl.BlockSpec((EC, 2), lambda k: (k, 0)),
        out_shape=jax.ShapeDtypeStruct((E_PAD, 2), _f32),
    )(e, s1, s2, ge, betae)


def kernel(x, edge_index, batch_index, patient_encoder_output, edgebindex,
           W1, b1, g1, beta1, W2, b2, g2, beta2, W_fc, b_fc, g_e, beta_e):
    pad0 = jnp.zeros((E_PAD - E,), jnp.int32)
    padn = jnp.full((E_PAD - E,), N, jnp.int32)
    src = jnp.concatenate([edge_index[0], pad0])
    dst = jnp.concatenate([edge_index[1], padn])
    dst_eh = jnp.concatenate([edge_index[1], pad0])
    ebi2d = jnp.concatenate([edgebindex, pad0])[:, None]
    ones = jnp.ones((CHUNK, DHH), _f32)
    zeros128 = jnp.zeros((N2, DHH), _f32)
    padc = jnp.zeros((DH, DHH - 2), _f32)
    wsrc128 = jnp.concatenate([W_fc[:DH], padc], axis=1)
    wdst128 = jnp.concatenate([W_fc[DH:2 * DH], padc], axis=1)
    wp128 = jnp.concatenate([W_fc[2 * DH:], padc[:P + P]], axis=1)
    iota2d = jnp.arange(NS * CHUNK, dtype=jnp.int32).reshape(NS, CHUNK)

    deg_a, deg_b = _deg_call(dst, ones, zeros128)
    t1a, t1b, dis = _stage1_call(deg_a, deg_b, x, W1)
    s1a, s1b = _conv_call(src, dst, t1a, t1b, zeros128)
    t2a, t2b = _stage2_call(s1a, s1b, dis, b1, g1, beta1, W2)
    s2a, s2b = _conv_call(src, dst, t2a, t2b, zeros128)
    ps128, pd128, pp128 = _stage3_call(s2a, s2b, dis, b2, g2, beta2,
                                       wsrc128, wdst128,
                                       patient_encoder_output, wp128)
    ge_rows = _edge_head_call(src, dst_eh, ps128, pd128, iota2d)
    e, s1, s2 = _stage4a_call(ge_rows, b_fc, ebi2d, pp128)
    return _stage4b_call(e, s1, s2, g_e, beta_e)[:E]
